# baseline (speedup 1.0000x reference)
import jax
import jax.numpy as jnp
from jax import lax
from jax.experimental import pallas as pl
from jax.experimental.pallas import tpu as pltpu
from jax.experimental.pallas import tpu_sc as plsc

DIMS = 512

_mesh = plsc.ScalarSubcoreMesh(axis_name="c", num_cores=1)


def _pe_lookup(xy_hbm, pe_x_hbm, pe_y_hbm, out_hbm, idx_s, sem_a, sem_b):
    c = lax.axis_index("c")

    @pl.when(c == 0)
    def _():
        pltpu.sync_copy(xy_hbm, idx_s)
        x = idx_s[0, 0]
        y = idx_s[1, 0]
        cp_x = pltpu.async_copy(
            pe_x_hbm.at[pl.ds(x, 1)], out_hbm.at[pl.ds(0, 1)], sem_a
        )
        cp_y = pltpu.async_copy(
            pe_y_hbm.at[pl.ds(y, 1)], out_hbm.at[pl.ds(1, 1)], sem_b
        )
        cp_x.wait()
        cp_y.wait()


_sc_call = pl.kernel(
    _pe_lookup,
    out_type=jax.ShapeDtypeStruct((2, DIMS), jnp.float32),
    mesh=_mesh,
    scratch_types=[
        pltpu.SMEM((2, 1), jnp.int32),
        pltpu.SemaphoreType.DMA,
        pltpu.SemaphoreType.DMA,
    ],
)


@jax.jit
def kernel(xy_tensor, pe_x, pe_y):
    xy = xy_tensor.reshape(2, 1)
    out = _sc_call(xy, pe_x, pe_y)
    return out.reshape(1, 2 * DIMS)
